# baseline (speedup 1.0000x reference)
import jax
import jax.numpy as jnp
from jax import lax
from jax.experimental import pallas as pl
from jax.experimental.pallas import tpu as pltpu

N_TOKENS = 65536
N_GROUPS = 8192
DIM = 256
BETA = 0.99
FACTOR = 1.0 - BETA

TILE_M = 512
TILE_N = 1024
N_SUB = N_GROUPS // TILE_N


def _assign_body(x_ref, gf_ref, out_ref):
    x = x_ref[...]

    def step(t, carry):
        best_v, best_i = carry
        gf = gf_ref[pl.ds(t * TILE_N, TILE_N), :]
        inv = lax.rsqrt(jnp.sum(gf * gf, axis=1, keepdims=True))
        gfn = gf * inv
        logits = lax.dot_general(x, gfn, (((1,), (1,)), ((), ())),
                                 preferred_element_type=jnp.float32)
        m = jnp.max(logits, axis=1, keepdims=True)
        iota = lax.broadcasted_iota(jnp.int32, (TILE_M, TILE_N), 1)
        loc = jnp.min(jnp.where(logits >= m, iota, TILE_N),
                      axis=1, keepdims=True)
        idx = loc + t * TILE_N
        better = m > best_v
        return jnp.where(better, m, best_v), jnp.where(better, idx, best_i)

    init = (jnp.full((TILE_M, 1), -jnp.inf, jnp.float32),
            jnp.zeros((TILE_M, 1), jnp.int32))
    _, best_i = lax.fori_loop(0, N_SUB, step, init)
    out_ref[...] = best_i


def _assign(x, gf):
    return pl.pallas_call(
        _assign_body,
        grid=(N_TOKENS // TILE_M,),
        in_specs=[
            pl.BlockSpec((TILE_M, DIM), lambda i: (i, 0)),
            pl.BlockSpec((N_GROUPS, DIM), lambda i: (0, 0)),
        ],
        out_specs=pl.BlockSpec((TILE_M, 1), lambda i: (i, 0)),
        out_shape=jax.ShapeDtypeStruct((N_TOKENS, 1), jnp.int32),
    )(x, gf)


GB = 512
TT = 2048
N_TT = N_TOKENS // TT


def _update_body(assign_ref, x_ref, gf_ref, out_ref, sums_ref, cnt_ref):
    i = pl.program_id(0)
    j = pl.program_id(1)

    @pl.when(j == 0)
    def _():
        sums_ref[...] = jnp.zeros((GB, DIM), jnp.float32)
        cnt_ref[...] = jnp.zeros((GB, 1), jnp.float32)

    a = assign_ref[...]
    gbase = i * GB
    iota = lax.broadcasted_iota(jnp.int32, (TT, GB), 1) + gbase
    onehot = jnp.where(a == iota, 1.0, 0.0).astype(jnp.float32)
    x = x_ref[...]
    sums_ref[...] += lax.dot_general(onehot, x, (((0,), (0,)), ((), ())),
                                     preferred_element_type=jnp.float32)
    ones = jnp.ones((TT, 1), jnp.float32)
    cnt_ref[...] += lax.dot_general(onehot, ones, (((0,), (0,)), ((), ())),
                                    preferred_element_type=jnp.float32)

    @pl.when(j == N_TT - 1)
    def _():
        cnt = cnt_ref[...]
        inv = jnp.where(cnt > 0.0, FACTOR / jnp.maximum(cnt, 1.0), 0.0)
        out_ref[...] = BETA * gf_ref[...] + sums_ref[...] * inv


def _update(assign2d, x, gf):
    return pl.pallas_call(
        _update_body,
        grid=(N_GROUPS // GB, N_TT),
        in_specs=[
            pl.BlockSpec((TT, 1), lambda i, j: (j, 0)),
            pl.BlockSpec((TT, DIM), lambda i, j: (j, 0)),
            pl.BlockSpec((GB, DIM), lambda i, j: (i, 0)),
        ],
        out_specs=pl.BlockSpec((GB, DIM), lambda i, j: (i, 0)),
        out_shape=jax.ShapeDtypeStruct((N_GROUPS, DIM), jnp.float32),
        scratch_shapes=[
            pltpu.VMEM((GB, DIM), jnp.float32),
            pltpu.VMEM((GB, 1), jnp.float32),
        ],
    )(assign2d, x, gf)


def kernel(x, group_features):
    assign2d = _assign(x, group_features)
    return _update(assign2d, x, group_features)


# speedup vs baseline: 1.0241x; 1.0241x over previous
import jax
import jax.numpy as jnp
from jax import lax
from jax.experimental import pallas as pl
from jax.experimental.pallas import tpu as pltpu

N_TOKENS = 65536
N_GROUPS = 8192
DIM = 256
BETA = 0.99
FACTOR = 1.0 - BETA

TILE_M = 512
TILE_N = 1024
N_SUB = N_GROUPS // TILE_N


def _assign_body(x_ref, gf_ref, out_ref):
    x = x_ref[...].astype(jnp.bfloat16)

    def step(t, carry):
        best_v, best_i = carry
        gf = gf_ref[pl.ds(t * TILE_N, TILE_N), :]
        inv = lax.rsqrt(jnp.sum(gf * gf, axis=1, keepdims=True))
        gfn = (gf * inv).astype(jnp.bfloat16)
        logits = lax.dot_general(x, gfn, (((1,), (1,)), ((), ())),
                                 preferred_element_type=jnp.float32)
        m = jnp.max(logits, axis=1, keepdims=True)
        iota = lax.broadcasted_iota(jnp.int32, (TILE_M, TILE_N), 1)
        loc = jnp.min(jnp.where(logits >= m, iota, TILE_N),
                      axis=1, keepdims=True)
        idx = loc + t * TILE_N
        better = m > best_v
        return jnp.where(better, m, best_v), jnp.where(better, idx, best_i)

    init = (jnp.full((TILE_M, 1), -jnp.inf, jnp.float32),
            jnp.zeros((TILE_M, 1), jnp.int32))
    _, best_i = lax.fori_loop(0, N_SUB, step, init)
    out_ref[...] = best_i


def _assign(x, gf):
    return pl.pallas_call(
        _assign_body,
        grid=(N_TOKENS // TILE_M,),
        in_specs=[
            pl.BlockSpec((TILE_M, DIM), lambda i: (i, 0)),
            pl.BlockSpec((N_GROUPS, DIM), lambda i: (0, 0)),
        ],
        out_specs=pl.BlockSpec((TILE_M, 1), lambda i: (i, 0)),
        out_shape=jax.ShapeDtypeStruct((N_TOKENS, 1), jnp.int32),
    )(x, gf)


GB = 512
TT = 2048
N_TT = N_TOKENS // TT


def _update_body(assign_ref, x_ref, gf_ref, out_ref, sums_ref, cnt_ref):
    i = pl.program_id(0)
    j = pl.program_id(1)

    @pl.when(j == 0)
    def _():
        sums_ref[...] = jnp.zeros((GB, DIM), jnp.float32)
        cnt_ref[...] = jnp.zeros((GB, 1), jnp.float32)

    a = assign_ref[...]
    gbase = i * GB
    iota = lax.broadcasted_iota(jnp.int32, (TT, GB), 1) + gbase
    onehot = jnp.where(a == iota, 1.0, 0.0).astype(jnp.bfloat16)
    x = x_ref[...].astype(jnp.bfloat16)
    sums_ref[...] += lax.dot_general(onehot, x, (((0,), (0,)), ((), ())),
                                     preferred_element_type=jnp.float32)
    ones = jnp.ones((TT, 1), jnp.bfloat16)
    cnt_ref[...] += lax.dot_general(onehot, ones, (((0,), (0,)), ((), ())),
                                    preferred_element_type=jnp.float32)

    @pl.when(j == N_TT - 1)
    def _():
        cnt = cnt_ref[...]
        inv = jnp.where(cnt > 0.0, FACTOR / jnp.maximum(cnt, 1.0), 0.0)
        out_ref[...] = BETA * gf_ref[...] + sums_ref[...] * inv


def _update(assign2d, x, gf):
    return pl.pallas_call(
        _update_body,
        grid=(N_GROUPS // GB, N_TT),
        in_specs=[
            pl.BlockSpec((TT, 1), lambda i, j: (j, 0)),
            pl.BlockSpec((TT, DIM), lambda i, j: (j, 0)),
            pl.BlockSpec((GB, DIM), lambda i, j: (i, 0)),
        ],
        out_specs=pl.BlockSpec((GB, DIM), lambda i, j: (i, 0)),
        out_shape=jax.ShapeDtypeStruct((N_GROUPS, DIM), jnp.float32),
        scratch_shapes=[
            pltpu.VMEM((GB, DIM), jnp.float32),
            pltpu.VMEM((GB, 1), jnp.float32),
        ],
    )(assign2d, x, gf)


def kernel(x, group_features):
    assign2d = _assign(x, group_features)
    return _update(assign2d, x, group_features)
